# baseline (speedup 1.0000x reference)
import jax
import jax.numpy as jnp
from jax import lax
from jax.experimental import pallas as pl
from jax.experimental.pallas import tpu as pltpu
from jax.experimental.pallas import tpu_sc as plsc

_INTERPRET = False

E = 8
D = 768
H = 768
T = 4096
BTR = 1024
BT = 256
NT = T // BT
NW = NT + E

NWORK = 32
TPW = T // NWORK


def _router_body(x_ref, wg_ref, eid_ref, rnk_ref, bases_ref, meta_ref,
                 loss_ref, acc_ref, blk_ref):
    i = pl.program_id(0)
    n = pl.num_programs(0)
    nblk = BTR // TPW
    logits = jnp.dot(x_ref[...], wg_ref[...], preferred_element_type=jnp.float32)
    lane = jax.lax.broadcasted_iota(jnp.int32, logits.shape, 1)
    logits = jnp.where(lane < E, logits, -jnp.inf)
    m = jnp.max(logits, axis=1, keepdims=True)
    eid = jnp.min(jnp.where(logits == m, lane, jnp.int32(2**30)), axis=1,
                  keepdims=True)
    onehot = jnp.where((lane == eid) & (lane < E), jnp.float32(1.0),
                       jnp.float32(0.0))
    ra = jax.lax.broadcasted_iota(jnp.int32, (TPW, TPW), 0)
    rb = jax.lax.broadcasted_iota(jnp.int32, (TPW, TPW), 1)
    tri = jnp.where(rb < ra, jnp.float32(1.0), jnp.float32(0.0))
    eid_rows, rnk_rows = [], []
    for j in range(nblk):
        sub = onehot[j * TPW:(j + 1) * TPW, :]
        blk_ref[pl.ds(i * nblk + j, 1), :] = jnp.sum(sub, axis=0,
                                                     keepdims=True)
        ranks = jnp.dot(tri, sub, preferred_element_type=jnp.float32)
        r_col = jnp.sum(ranks * sub, axis=1, keepdims=True).astype(jnp.int32)
        e_col = eid[j * TPW:(j + 1) * TPW, :]
        eid_rows.append(jnp.transpose(e_col, (1, 0)))
        rnk_rows.append(jnp.transpose(r_col, (1, 0)))
    eid_ref[...] = jnp.concatenate(eid_rows, 0)
    rnk_ref[...] = jnp.concatenate(rnk_rows, 0)

    @pl.when(i == 0)
    def _():
        acc_ref[...] = jnp.zeros_like(acc_ref)

    acc_ref[...] += jnp.sum(onehot, axis=0, keepdims=True)

    @pl.when(i == n - 1)
    def _():
        c = acc_ref[...]
        lane1 = lane[:1, :]
        mean = jnp.sum(jnp.where(lane1 < E, c, 0.0)) / jnp.float32(E)
        dev = jnp.where(lane1 < E, c - mean, 0.0)
        var = jnp.sum(dev * dev) / jnp.float32(E - 1)
        loss_ref[...] = jnp.full((1, 1), var / (mean * mean + 1e-10),
                                 jnp.float32)
        ua = jax.lax.broadcasted_iota(jnp.int32, (128, 128), 0)
        ub = jax.lax.broadcasted_iota(jnp.int32, (128, 128), 1)
        triu = jnp.where(ua < ub, jnp.float32(1.0), jnp.float32(0.0))
        offs_row = jnp.dot(c, triu, preferred_element_type=jnp.float32,
                           precision=jax.lax.Precision.HIGHEST)
        wa = jax.lax.broadcasted_iota(jnp.int32, (NWORK, NWORK), 0)
        wb = jax.lax.broadcasted_iota(jnp.int32, (NWORK, NWORK), 1)
        tri32 = jnp.where(wb < wa, jnp.float32(1.0), jnp.float32(0.0))
        pre = jnp.dot(tri32, blk_ref[...], preferred_element_type=jnp.float32,
                      precision=jax.lax.Precision.HIGHEST)
        bases_ref[...] = (pre + offs_row).astype(jnp.int32)
        interior = (lane1 >= 1) & (lane1 < E)
        starts = [jnp.float32(0.0)]
        es_list, ee_list = [], []
        for t in range(NT):
            es = jnp.sum(jnp.where(interior & (offs_row <= t * BT), 1.0, 0.0))
            ee = jnp.sum(jnp.where(
                interior & (offs_row <= t * BT + (BT - 1)), 1.0, 0.0))
            es_list.append(es)
            ee_list.append(ee)
            starts.append(starts[-1] + (ee - es + 1.0))
        ntot = starts[NT]
        wlane = lane1.astype(jnp.float32)
        wc = jnp.minimum(wlane, ntot - 1.0)
        t_of_w = jnp.zeros_like(wlane)
        for t in range(NT):
            t_of_w += jnp.where(starts[t] <= wc, 1.0, 0.0)
        t_of_w -= 1.0
        e_of_w = jnp.zeros_like(wlane)
        for t in range(NT):
            sel = t_of_w == jnp.float32(t)
            e_of_w = jnp.where(sel, es_list[t] + (wc - starts[t]), e_of_w)
        meta_ref[0:1, :] = offs_row.astype(jnp.int32)
        meta_ref[1:2, :] = t_of_w.astype(jnp.int32)
        meta_ref[2:3, :] = e_of_w.astype(jnp.int32)
        for r in range(3, 8):
            meta_ref[r:r + 1, :] = jnp.zeros((1, 128), jnp.int32)


def _router(x, wg_pad):
    return pl.pallas_call(
        _router_body,
        grid=(T // BTR,),
        in_specs=[
            pl.BlockSpec((BTR, D), lambda i: (i, 0)),
            pl.BlockSpec((D, 128), lambda i: (0, 0)),
        ],
        out_specs=[
            pl.BlockSpec((BTR // TPW, 128), lambda i: (i, 0)),
            pl.BlockSpec((BTR // TPW, 128), lambda i: (i, 0)),
            pl.BlockSpec((NWORK, 128), lambda i: (0, 0)),
            pl.BlockSpec((8, 128), lambda i: (0, 0)),
            pl.BlockSpec((1, 1), lambda i: (0, 0)),
        ],
        out_shape=[
            jax.ShapeDtypeStruct((NWORK, 128), jnp.int32),
            jax.ShapeDtypeStruct((NWORK, 128), jnp.int32),
            jax.ShapeDtypeStruct((NWORK, 128), jnp.int32),
            jax.ShapeDtypeStruct((8, 128), jnp.int32),
            jax.ShapeDtypeStruct((1, 1), jnp.float32),
        ],
        scratch_shapes=[pltpu.VMEM((1, 128), jnp.float32),
                        pltpu.VMEM((NWORK, 128), jnp.float32)],
        interpret=_INTERPRET,
    )(x, wg_pad)


def _sc_dispatch_body(eid_hbm, rnk_hbm, bases_hbm, x_hbm, dest_hbm, xs_hbm,
                      eid_v, rnk_v, d0_v, d1_v, base_v, x0_v, x1_v,
                      sem_a, sem_b, sem_c, sem_d):
    hw = TPW // 2
    wid = lax.axis_index("s") * 2 + lax.axis_index("c")
    tok0 = wid * TPW
    l0 = pltpu.async_copy(x_hbm.at[pl.ds(tok0, hw)], x0_v, sem_a)
    l1 = pltpu.async_copy(x_hbm.at[pl.ds(tok0 + hw, hw)], x1_v, sem_b)
    m0 = pltpu.async_copy(eid_hbm.at[wid], eid_v, sem_c)
    m1 = pltpu.async_copy(rnk_hbm.at[wid], rnk_v, sem_d)
    pltpu.sync_copy(bases_hbm.at[wid], base_v)
    m0.wait()
    m1.wait()
    for ch in range(TPW // 16):
        v = eid_v[pl.ds(ch * 16, 16)]
        b = plsc.load_gather(base_v, [v])
        d = b + rnk_v[pl.ds(ch * 16, 16)]
        if ch < hw // 16:
            d0_v[pl.ds(ch * 16, 16)] = d
        else:
            d1_v[pl.ds(ch * 16 - hw, 16)] = d
    l0.wait()
    s0 = pltpu.async_copy(x0_v, xs_hbm.at[d0_v], sem_c)
    l1.wait()
    s1 = pltpu.async_copy(x1_v, xs_hbm.at[d1_v], sem_d)
    pltpu.sync_copy(d0_v, dest_hbm.at[pl.ds(tok0, hw)])
    pltpu.sync_copy(d1_v, dest_hbm.at[pl.ds(tok0 + hw, hw)])
    s0.wait()
    s1.wait()


def _sc_dispatch(eid, rnk, bases, x):
    mesh = plsc.VectorSubcoreMesh(core_axis_name="c", subcore_axis_name="s")
    return pl.kernel(
        _sc_dispatch_body,
        out_type=[
            jax.ShapeDtypeStruct((T,), jnp.int32),
            jax.ShapeDtypeStruct((T, D), jnp.float32),
        ],
        mesh=mesh,
        scratch_types=[
            pltpu.VMEM((TPW,), jnp.int32),
            pltpu.VMEM((TPW,), jnp.int32),
            pltpu.VMEM((TPW // 2,), jnp.int32),
            pltpu.VMEM((TPW // 2,), jnp.int32),
            pltpu.VMEM((128,), jnp.int32),
            pltpu.VMEM((TPW // 2, D), jnp.float32),
            pltpu.VMEM((TPW // 2, D), jnp.float32),
            pltpu.SemaphoreType.DMA,
            pltpu.SemaphoreType.DMA,
            pltpu.SemaphoreType.DMA,
            pltpu.SemaphoreType.DMA,
        ],
        compiler_params=pltpu.CompilerParams(needs_layout_passes=False),
        interpret=_INTERPRET,
    )(eid, rnk, bases, x)


def _sc_combine_body(dest_hbm, y_hbm, out_hbm, d0_v, d1_v, y0_v, y1_v,
                     sem_a, sem_b):
    hw = TPW // 2
    wid = lax.axis_index("s") * 2 + lax.axis_index("c")
    tok0 = wid * TPW
    ld0 = pltpu.async_copy(dest_hbm.at[pl.ds(tok0, hw)], d0_v, sem_a)
    ld1 = pltpu.async_copy(dest_hbm.at[pl.ds(tok0 + hw, hw)], d1_v, sem_b)
    ld0.wait()
    g0 = pltpu.async_copy(y_hbm.at[d0_v], y0_v, sem_a)
    ld1.wait()
    g1 = pltpu.async_copy(y_hbm.at[d1_v], y1_v, sem_b)
    g0.wait()
    pltpu.sync_copy(y0_v, out_hbm.at[pl.ds(tok0, hw)])
    g1.wait()
    pltpu.sync_copy(y1_v, out_hbm.at[pl.ds(tok0 + hw, hw)])


def _sc_combine(dest, y_sorted):
    mesh = plsc.VectorSubcoreMesh(core_axis_name="c", subcore_axis_name="s")
    return pl.kernel(
        _sc_combine_body,
        out_type=jax.ShapeDtypeStruct((T, D), jnp.float32),
        mesh=mesh,
        scratch_types=[
            pltpu.VMEM((TPW // 2,), jnp.int32),
            pltpu.VMEM((TPW // 2,), jnp.int32),
            pltpu.VMEM((TPW // 2, D), jnp.float32),
            pltpu.VMEM((TPW // 2, D), jnp.float32),
            pltpu.SemaphoreType.DMA,
            pltpu.SemaphoreType.DMA,
        ],
        compiler_params=pltpu.CompilerParams(needs_layout_passes=False),
        interpret=_INTERPRET,
    )(dest, y_sorted)


def _gmm_body(meta_ref, x_ref, w1_ref, b1_ref, w2_ref, b2_ref, y_ref):
    w = pl.program_id(0)
    t = meta_ref[1, w]
    e = meta_ref[2, w]
    s = meta_ref[0, e]
    epos = meta_ref[0, e + 1]
    rows = t * BT + jax.lax.broadcasted_iota(jnp.int32, (BT, 1), 0)
    mask = (rows >= s) & (rows < epos)
    h = jnp.maximum(
        jnp.dot(x_ref[...], w1_ref[0].astype(jnp.bfloat16),
                preferred_element_type=jnp.float32) + b1_ref[0], 0.0)
    y = jnp.dot(h.astype(jnp.bfloat16), w2_ref[0].astype(jnp.bfloat16),
                preferred_element_type=jnp.float32) + b2_ref[0]
    y_ref[...] = jnp.where(mask, y, y_ref[...])


def _gmm(meta, x_sorted, W1, b1, W2, b2):
    grid_spec = pltpu.PrefetchScalarGridSpec(
        num_scalar_prefetch=1,
        grid=(NW,),
        in_specs=[
            pl.BlockSpec((BT, D), lambda w, mr: (mr[1, w], 0)),
            pl.BlockSpec((1, D, H), lambda w, mr: (mr[2, w], 0, 0)),
            pl.BlockSpec((1, 1, H), lambda w, mr: (mr[2, w], 0, 0)),
            pl.BlockSpec((1, H, D), lambda w, mr: (mr[2, w], 0, 0)),
            pl.BlockSpec((1, 1, D), lambda w, mr: (mr[2, w], 0, 0)),
        ],
        out_specs=pl.BlockSpec((BT, D), lambda w, mr: (mr[1, w], 0)),
    )
    return pl.pallas_call(
        _gmm_body,
        grid_spec=grid_spec,
        out_shape=jax.ShapeDtypeStruct((T, D), jnp.float32),
        compiler_params=pltpu.CompilerParams(
            dimension_semantics=("arbitrary",)),
        interpret=_INTERPRET,
    )(meta, x_sorted, W1,
      b1.reshape(E, 1, H), W2, b2.reshape(E, 1, D))


def kernel(x, Wg, W1, b1, W2, b2):
    wg_pad = jnp.zeros((D, 128), jnp.float32).at[:, :E].set(Wg)
    eid, rnk, bases, meta, loss11 = _router(x, wg_pad)
    loss = loss11[0, 0]

    dest, x_sorted = _sc_dispatch(eid, rnk, bases, x)
    y_sorted = _gmm(meta, x_sorted, W1, b1, W2, b2)
    out = _sc_combine(dest, y_sorted)
    return out, loss, loss


# speedup vs baseline: 1.0436x; 1.0436x over previous
import jax
import jax.numpy as jnp
from jax import lax
from jax.experimental import pallas as pl
from jax.experimental.pallas import tpu as pltpu
from jax.experimental.pallas import tpu_sc as plsc

_INTERPRET = False

E = 8
D = 768
H = 768
T = 4096
BTR = 1024
BT = 512
NT = T // BT
NW = NT + E

NWORK = 32
TPW = T // NWORK


def _router_body(x_ref, wg_ref, eid_ref, rnk_ref, bases_ref, meta_ref,
                 loss_ref, acc_ref, blk_ref):
    i = pl.program_id(0)
    n = pl.num_programs(0)
    nblk = BTR // TPW
    logits = jnp.dot(x_ref[...], wg_ref[...], preferred_element_type=jnp.float32)
    lane = jax.lax.broadcasted_iota(jnp.int32, logits.shape, 1)
    logits = jnp.where(lane < E, logits, -jnp.inf)
    m = jnp.max(logits, axis=1, keepdims=True)
    eid = jnp.min(jnp.where(logits == m, lane, jnp.int32(2**30)), axis=1,
                  keepdims=True)
    onehot = jnp.where((lane == eid) & (lane < E), jnp.float32(1.0),
                       jnp.float32(0.0))
    ra = jax.lax.broadcasted_iota(jnp.int32, (TPW, TPW), 0)
    rb = jax.lax.broadcasted_iota(jnp.int32, (TPW, TPW), 1)
    tri = jnp.where(rb < ra, jnp.float32(1.0), jnp.float32(0.0))
    eid_rows, rnk_rows = [], []
    for j in range(nblk):
        sub = onehot[j * TPW:(j + 1) * TPW, :]
        blk_ref[pl.ds(i * nblk + j, 1), :] = jnp.sum(sub, axis=0,
                                                     keepdims=True)
        ranks = jnp.dot(tri, sub, preferred_element_type=jnp.float32)
        r_col = jnp.sum(ranks * sub, axis=1, keepdims=True).astype(jnp.int32)
        e_col = eid[j * TPW:(j + 1) * TPW, :]
        eid_rows.append(jnp.transpose(e_col, (1, 0)))
        rnk_rows.append(jnp.transpose(r_col, (1, 0)))
    eid_ref[...] = jnp.concatenate(eid_rows, 0)
    rnk_ref[...] = jnp.concatenate(rnk_rows, 0)

    @pl.when(i == 0)
    def _():
        acc_ref[...] = jnp.zeros_like(acc_ref)

    acc_ref[...] += jnp.sum(onehot, axis=0, keepdims=True)

    @pl.when(i == n - 1)
    def _():
        c = acc_ref[...]
        lane1 = lane[:1, :]
        mean = jnp.sum(jnp.where(lane1 < E, c, 0.0)) / jnp.float32(E)
        dev = jnp.where(lane1 < E, c - mean, 0.0)
        var = jnp.sum(dev * dev) / jnp.float32(E - 1)
        loss_ref[...] = jnp.full((1, 1), var / (mean * mean + 1e-10),
                                 jnp.float32)
        ua = jax.lax.broadcasted_iota(jnp.int32, (128, 128), 0)
        ub = jax.lax.broadcasted_iota(jnp.int32, (128, 128), 1)
        triu = jnp.where(ua < ub, jnp.float32(1.0), jnp.float32(0.0))
        offs_row = jnp.dot(c, triu, preferred_element_type=jnp.float32,
                           precision=jax.lax.Precision.HIGHEST)
        wa = jax.lax.broadcasted_iota(jnp.int32, (NWORK, NWORK), 0)
        wb = jax.lax.broadcasted_iota(jnp.int32, (NWORK, NWORK), 1)
        tri32 = jnp.where(wb < wa, jnp.float32(1.0), jnp.float32(0.0))
        pre = jnp.dot(tri32, blk_ref[...], preferred_element_type=jnp.float32,
                      precision=jax.lax.Precision.HIGHEST)
        bases_ref[...] = (pre + offs_row).astype(jnp.int32)
        interior = (lane1 >= 1) & (lane1 < E)
        starts = [jnp.float32(0.0)]
        es_list, ee_list = [], []
        for t in range(NT):
            es = jnp.sum(jnp.where(interior & (offs_row <= t * BT), 1.0, 0.0))
            ee = jnp.sum(jnp.where(
                interior & (offs_row <= t * BT + (BT - 1)), 1.0, 0.0))
            es_list.append(es)
            ee_list.append(ee)
            starts.append(starts[-1] + (ee - es + 1.0))
        ntot = starts[NT]
        wlane = lane1.astype(jnp.float32)
        wc = jnp.minimum(wlane, ntot - 1.0)
        t_of_w = jnp.zeros_like(wlane)
        for t in range(NT):
            t_of_w += jnp.where(starts[t] <= wc, 1.0, 0.0)
        t_of_w -= 1.0
        e_of_w = jnp.zeros_like(wlane)
        for t in range(NT):
            sel = t_of_w == jnp.float32(t)
            e_of_w = jnp.where(sel, es_list[t] + (wc - starts[t]), e_of_w)
        lane_is9 = lane1 == 9
        meta0 = jnp.where(lane_is9, ntot, offs_row)
        meta_ref[0:1, :] = meta0.astype(jnp.int32)
        meta_ref[1:2, :] = t_of_w.astype(jnp.int32)
        meta_ref[2:3, :] = e_of_w.astype(jnp.int32)
        for r in range(3, 8):
            meta_ref[r:r + 1, :] = jnp.zeros((1, 128), jnp.int32)


def _router(x, wg_pad):
    return pl.pallas_call(
        _router_body,
        grid=(T // BTR,),
        in_specs=[
            pl.BlockSpec((BTR, D), lambda i: (i, 0)),
            pl.BlockSpec((D, 128), lambda i: (0, 0)),
        ],
        out_specs=[
            pl.BlockSpec((BTR // TPW, 128), lambda i: (i, 0)),
            pl.BlockSpec((BTR // TPW, 128), lambda i: (i, 0)),
            pl.BlockSpec((NWORK, 128), lambda i: (0, 0)),
            pl.BlockSpec((8, 128), lambda i: (0, 0)),
            pl.BlockSpec((1, 1), lambda i: (0, 0)),
        ],
        out_shape=[
            jax.ShapeDtypeStruct((NWORK, 128), jnp.int32),
            jax.ShapeDtypeStruct((NWORK, 128), jnp.int32),
            jax.ShapeDtypeStruct((NWORK, 128), jnp.int32),
            jax.ShapeDtypeStruct((8, 128), jnp.int32),
            jax.ShapeDtypeStruct((1, 1), jnp.float32),
        ],
        scratch_shapes=[pltpu.VMEM((1, 128), jnp.float32),
                        pltpu.VMEM((NWORK, 128), jnp.float32)],
        interpret=_INTERPRET,
    )(x, wg_pad)


def _sc_dispatch_body(eid_hbm, rnk_hbm, bases_hbm, x_hbm, dest_hbm, xs_hbm,
                      eid_v, rnk_v, d0_v, d1_v, base_v, x0_v, x1_v,
                      sem_a, sem_b, sem_c, sem_d):
    hw = TPW // 2
    wid = lax.axis_index("s") * 2 + lax.axis_index("c")
    tok0 = wid * TPW
    l0 = pltpu.async_copy(x_hbm.at[pl.ds(tok0, hw)], x0_v, sem_a)
    l1 = pltpu.async_copy(x_hbm.at[pl.ds(tok0 + hw, hw)], x1_v, sem_b)
    m0 = pltpu.async_copy(eid_hbm.at[wid], eid_v, sem_c)
    m1 = pltpu.async_copy(rnk_hbm.at[wid], rnk_v, sem_d)
    pltpu.sync_copy(bases_hbm.at[wid], base_v)
    m0.wait()
    m1.wait()
    for ch in range(TPW // 16):
        v = eid_v[pl.ds(ch * 16, 16)]
        b = plsc.load_gather(base_v, [v])
        d = b + rnk_v[pl.ds(ch * 16, 16)]
        if ch < hw // 16:
            d0_v[pl.ds(ch * 16, 16)] = d
        else:
            d1_v[pl.ds(ch * 16 - hw, 16)] = d
    l0.wait()
    s0 = pltpu.async_copy(x0_v, xs_hbm.at[d0_v], sem_c)
    l1.wait()
    s1 = pltpu.async_copy(x1_v, xs_hbm.at[d1_v], sem_d)
    pltpu.sync_copy(d0_v, dest_hbm.at[pl.ds(tok0, hw)])
    pltpu.sync_copy(d1_v, dest_hbm.at[pl.ds(tok0 + hw, hw)])
    s0.wait()
    s1.wait()


def _sc_dispatch(eid, rnk, bases, x):
    mesh = plsc.VectorSubcoreMesh(core_axis_name="c", subcore_axis_name="s")
    return pl.kernel(
        _sc_dispatch_body,
        out_type=[
            jax.ShapeDtypeStruct((T,), jnp.int32),
            jax.ShapeDtypeStruct((T, D), jnp.float32),
        ],
        mesh=mesh,
        scratch_types=[
            pltpu.VMEM((TPW,), jnp.int32),
            pltpu.VMEM((TPW,), jnp.int32),
            pltpu.VMEM((TPW // 2,), jnp.int32),
            pltpu.VMEM((TPW // 2,), jnp.int32),
            pltpu.VMEM((128,), jnp.int32),
            pltpu.VMEM((TPW // 2, D), jnp.float32),
            pltpu.VMEM((TPW // 2, D), jnp.float32),
            pltpu.SemaphoreType.DMA,
            pltpu.SemaphoreType.DMA,
            pltpu.SemaphoreType.DMA,
            pltpu.SemaphoreType.DMA,
        ],
        compiler_params=pltpu.CompilerParams(needs_layout_passes=False),
        interpret=_INTERPRET,
    )(eid, rnk, bases, x)


def _sc_combine_body(dest_hbm, y_hbm, out_hbm, d0_v, d1_v, y0_v, y1_v,
                     sem_a, sem_b):
    hw = TPW // 2
    wid = lax.axis_index("s") * 2 + lax.axis_index("c")
    tok0 = wid * TPW
    ld0 = pltpu.async_copy(dest_hbm.at[pl.ds(tok0, hw)], d0_v, sem_a)
    ld1 = pltpu.async_copy(dest_hbm.at[pl.ds(tok0 + hw, hw)], d1_v, sem_b)
    ld0.wait()
    g0 = pltpu.async_copy(y_hbm.at[d0_v], y0_v, sem_a)
    ld1.wait()
    g1 = pltpu.async_copy(y_hbm.at[d1_v], y1_v, sem_b)
    g0.wait()
    pltpu.sync_copy(y0_v, out_hbm.at[pl.ds(tok0, hw)])
    g1.wait()
    pltpu.sync_copy(y1_v, out_hbm.at[pl.ds(tok0 + hw, hw)])


def _sc_combine(dest, y_sorted):
    mesh = plsc.VectorSubcoreMesh(core_axis_name="c", subcore_axis_name="s")
    return pl.kernel(
        _sc_combine_body,
        out_type=jax.ShapeDtypeStruct((T, D), jnp.float32),
        mesh=mesh,
        scratch_types=[
            pltpu.VMEM((TPW // 2,), jnp.int32),
            pltpu.VMEM((TPW // 2,), jnp.int32),
            pltpu.VMEM((TPW // 2, D), jnp.float32),
            pltpu.VMEM((TPW // 2, D), jnp.float32),
            pltpu.SemaphoreType.DMA,
            pltpu.SemaphoreType.DMA,
        ],
        compiler_params=pltpu.CompilerParams(needs_layout_passes=False),
        interpret=_INTERPRET,
    )(dest, y_sorted)


def _gmm_body(meta_ref, x_ref, w1_ref, b1_ref, w2_ref, b2_ref, y_ref):
    w = pl.program_id(0)

    @pl.when(w < meta_ref[0, 9])
    def _():
        t = meta_ref[1, w]
        e = meta_ref[2, w]
        s = meta_ref[0, e]
        epos = meta_ref[0, e + 1]
        rows = t * BT + jax.lax.broadcasted_iota(jnp.int32, (BT, 1), 0)
        mask = (rows >= s) & (rows < epos)
        h = jnp.maximum(
            jnp.dot(x_ref[...], w1_ref[0].astype(jnp.bfloat16),
                    preferred_element_type=jnp.float32) + b1_ref[0], 0.0)
        y = jnp.dot(h.astype(jnp.bfloat16), w2_ref[0].astype(jnp.bfloat16),
                    preferred_element_type=jnp.float32) + b2_ref[0]
        y_ref[...] = jnp.where(mask, y, y_ref[...])


def _gmm(meta, x_sorted, W1, b1, W2, b2):
    grid_spec = pltpu.PrefetchScalarGridSpec(
        num_scalar_prefetch=1,
        grid=(NW,),
        in_specs=[
            pl.BlockSpec((BT, D), lambda w, mr: (mr[1, w], 0)),
            pl.BlockSpec((1, D, H), lambda w, mr: (mr[2, w], 0, 0)),
            pl.BlockSpec((1, 1, H), lambda w, mr: (mr[2, w], 0, 0)),
            pl.BlockSpec((1, H, D), lambda w, mr: (mr[2, w], 0, 0)),
            pl.BlockSpec((1, 1, D), lambda w, mr: (mr[2, w], 0, 0)),
        ],
        out_specs=pl.BlockSpec((BT, D), lambda w, mr: (mr[1, w], 0)),
    )
    return pl.pallas_call(
        _gmm_body,
        grid_spec=grid_spec,
        out_shape=jax.ShapeDtypeStruct((T, D), jnp.float32),
        compiler_params=pltpu.CompilerParams(
            dimension_semantics=("arbitrary",)),
        interpret=_INTERPRET,
    )(meta, x_sorted, W1,
      b1.reshape(E, 1, H), W2, b2.reshape(E, 1, D))


def kernel(x, Wg, W1, b1, W2, b2):
    wg_pad = jnp.zeros((D, 128), jnp.float32).at[:, :E].set(Wg)
    eid, rnk, bases, meta, loss11 = _router(x, wg_pad)
    loss = loss11[0, 0]

    dest, x_sorted = _sc_dispatch(eid, rnk, bases, x)
    y_sorted = _gmm(meta, x_sorted, W1, b1, W2, b2)
    out = _sc_combine(dest, y_sorted)
    return out, loss, loss


# speedup vs baseline: 1.0542x; 1.0101x over previous
import jax
import jax.numpy as jnp
from jax import lax
from jax.experimental import pallas as pl
from jax.experimental.pallas import tpu as pltpu
from jax.experimental.pallas import tpu_sc as plsc

_INTERPRET = False

E = 8
D = 768
H = 768
T = 4096
BTR = 2048
BT = 512
NT = T // BT
NW = NT + E

NWORK = 32
TPW = T // NWORK


def _router_body(x_ref, wg_ref, eid_ref, rnk_ref, bases_ref, meta_ref,
                 loss_ref, acc_ref, blk_ref):
    i = pl.program_id(0)
    n = pl.num_programs(0)
    nblk = BTR // TPW
    logits = jnp.dot(x_ref[...], wg_ref[...], preferred_element_type=jnp.float32)
    lane = jax.lax.broadcasted_iota(jnp.int32, logits.shape, 1)
    logits = jnp.where(lane < E, logits, -jnp.inf)
    m = jnp.max(logits, axis=1, keepdims=True)
    eid = jnp.min(jnp.where(logits == m, lane, jnp.int32(2**30)), axis=1,
                  keepdims=True)
    onehot = jnp.where((lane == eid) & (lane < E), jnp.float32(1.0),
                       jnp.float32(0.0))
    ra = jax.lax.broadcasted_iota(jnp.int32, (TPW, TPW), 0)
    rb = jax.lax.broadcasted_iota(jnp.int32, (TPW, TPW), 1)
    tri = jnp.where(rb < ra, jnp.float32(1.0), jnp.float32(0.0))
    eid_rows, rnk_rows = [], []
    for j in range(nblk):
        sub = onehot[j * TPW:(j + 1) * TPW, :]
        blk_ref[pl.ds(i * nblk + j, 1), :] = jnp.sum(sub, axis=0,
                                                     keepdims=True)
        ranks = jnp.dot(tri, sub, preferred_element_type=jnp.float32)
        r_col = jnp.sum(ranks * sub, axis=1, keepdims=True).astype(jnp.int32)
        e_col = eid[j * TPW:(j + 1) * TPW, :]
        eid_rows.append(jnp.transpose(e_col, (1, 0)))
        rnk_rows.append(jnp.transpose(r_col, (1, 0)))
    eid_ref[...] = jnp.concatenate(eid_rows, 0)
    rnk_ref[...] = jnp.concatenate(rnk_rows, 0)

    @pl.when(i == 0)
    def _():
        acc_ref[...] = jnp.zeros_like(acc_ref)

    acc_ref[...] += jnp.sum(onehot, axis=0, keepdims=True)

    @pl.when(i == n - 1)
    def _():
        c = acc_ref[...]
        lane1 = lane[:1, :]
        mean = jnp.sum(jnp.where(lane1 < E, c, 0.0)) / jnp.float32(E)
        dev = jnp.where(lane1 < E, c - mean, 0.0)
        var = jnp.sum(dev * dev) / jnp.float32(E - 1)
        loss_ref[...] = jnp.full((1, 1), var / (mean * mean + 1e-10),
                                 jnp.float32)
        ua = jax.lax.broadcasted_iota(jnp.int32, (128, 128), 0)
        ub = jax.lax.broadcasted_iota(jnp.int32, (128, 128), 1)
        triu = jnp.where(ua < ub, jnp.float32(1.0), jnp.float32(0.0))
        offs_row = jnp.dot(c, triu, preferred_element_type=jnp.float32,
                           precision=jax.lax.Precision.HIGHEST)
        wa = jax.lax.broadcasted_iota(jnp.int32, (NWORK, NWORK), 0)
        wb = jax.lax.broadcasted_iota(jnp.int32, (NWORK, NWORK), 1)
        tri32 = jnp.where(wb < wa, jnp.float32(1.0), jnp.float32(0.0))
        pre = jnp.dot(tri32, blk_ref[...], preferred_element_type=jnp.float32,
                      precision=jax.lax.Precision.HIGHEST)
        bases_ref[...] = (pre + offs_row).astype(jnp.int32)
        interior = (lane1 >= 1) & (lane1 < E)
        starts = [jnp.float32(0.0)]
        es_list, ee_list = [], []
        for t in range(NT):
            es = jnp.sum(jnp.where(interior & (offs_row <= t * BT), 1.0, 0.0))
            ee = jnp.sum(jnp.where(
                interior & (offs_row <= t * BT + (BT - 1)), 1.0, 0.0))
            es_list.append(es)
            ee_list.append(ee)
            starts.append(starts[-1] + (ee - es + 1.0))
        ntot = starts[NT]
        wlane = lane1.astype(jnp.float32)
        wc = jnp.minimum(wlane, ntot - 1.0)
        t_of_w = jnp.zeros_like(wlane)
        for t in range(NT):
            t_of_w += jnp.where(starts[t] <= wc, 1.0, 0.0)
        t_of_w -= 1.0
        e_of_w = jnp.zeros_like(wlane)
        for t in range(NT):
            sel = t_of_w == jnp.float32(t)
            e_of_w = jnp.where(sel, es_list[t] + (wc - starts[t]), e_of_w)
        lane_is9 = lane1 == 9
        meta0 = jnp.where(lane_is9, ntot, offs_row)
        meta_ref[0:1, :] = meta0.astype(jnp.int32)
        meta_ref[1:2, :] = t_of_w.astype(jnp.int32)
        meta_ref[2:3, :] = e_of_w.astype(jnp.int32)
        for r in range(3, 8):
            meta_ref[r:r + 1, :] = jnp.zeros((1, 128), jnp.int32)


def _router(x, wg_pad):
    return pl.pallas_call(
        _router_body,
        grid=(T // BTR,),
        in_specs=[
            pl.BlockSpec((BTR, D), lambda i: (i, 0)),
            pl.BlockSpec((D, 128), lambda i: (0, 0)),
        ],
        out_specs=[
            pl.BlockSpec((BTR // TPW, 128), lambda i: (i, 0)),
            pl.BlockSpec((BTR // TPW, 128), lambda i: (i, 0)),
            pl.BlockSpec((NWORK, 128), lambda i: (0, 0)),
            pl.BlockSpec((8, 128), lambda i: (0, 0)),
            pl.BlockSpec((1, 1), lambda i: (0, 0)),
        ],
        out_shape=[
            jax.ShapeDtypeStruct((NWORK, 128), jnp.int32),
            jax.ShapeDtypeStruct((NWORK, 128), jnp.int32),
            jax.ShapeDtypeStruct((NWORK, 128), jnp.int32),
            jax.ShapeDtypeStruct((8, 128), jnp.int32),
            jax.ShapeDtypeStruct((1, 1), jnp.float32),
        ],
        scratch_shapes=[pltpu.VMEM((1, 128), jnp.float32),
                        pltpu.VMEM((NWORK, 128), jnp.float32)],
        interpret=_INTERPRET,
    )(x, wg_pad)


def _sc_dispatch_body(eid_hbm, rnk_hbm, bases_hbm, x_hbm, dest_hbm, xs_hbm,
                      eid_v, rnk_v, d0_v, d1_v, base_v, x0_v, x1_v,
                      sem_a, sem_b, sem_c, sem_d):
    hw = TPW // 2
    wid = lax.axis_index("s") * 2 + lax.axis_index("c")
    tok0 = wid * TPW
    l0 = pltpu.async_copy(x_hbm.at[pl.ds(tok0, hw)], x0_v, sem_a)
    l1 = pltpu.async_copy(x_hbm.at[pl.ds(tok0 + hw, hw)], x1_v, sem_b)
    m0 = pltpu.async_copy(eid_hbm.at[wid], eid_v, sem_c)
    m1 = pltpu.async_copy(rnk_hbm.at[wid], rnk_v, sem_d)
    pltpu.sync_copy(bases_hbm.at[wid], base_v)
    m0.wait()
    m1.wait()
    for ch in range(TPW // 16):
        v = eid_v[pl.ds(ch * 16, 16)]
        b = plsc.load_gather(base_v, [v])
        d = b + rnk_v[pl.ds(ch * 16, 16)]
        if ch < hw // 16:
            d0_v[pl.ds(ch * 16, 16)] = d
        else:
            d1_v[pl.ds(ch * 16 - hw, 16)] = d
    l0.wait()
    s0 = pltpu.async_copy(x0_v, xs_hbm.at[d0_v], sem_c)
    l1.wait()
    s1 = pltpu.async_copy(x1_v, xs_hbm.at[d1_v], sem_d)
    pltpu.sync_copy(d0_v, dest_hbm.at[pl.ds(tok0, hw)])
    pltpu.sync_copy(d1_v, dest_hbm.at[pl.ds(tok0 + hw, hw)])
    s0.wait()
    s1.wait()


def _sc_dispatch(eid, rnk, bases, x):
    mesh = plsc.VectorSubcoreMesh(core_axis_name="c", subcore_axis_name="s")
    return pl.kernel(
        _sc_dispatch_body,
        out_type=[
            jax.ShapeDtypeStruct((T,), jnp.int32),
            jax.ShapeDtypeStruct((T, D), jnp.float32),
        ],
        mesh=mesh,
        scratch_types=[
            pltpu.VMEM((TPW,), jnp.int32),
            pltpu.VMEM((TPW,), jnp.int32),
            pltpu.VMEM((TPW // 2,), jnp.int32),
            pltpu.VMEM((TPW // 2,), jnp.int32),
            pltpu.VMEM((128,), jnp.int32),
            pltpu.VMEM((TPW // 2, D), jnp.float32),
            pltpu.VMEM((TPW // 2, D), jnp.float32),
            pltpu.SemaphoreType.DMA,
            pltpu.SemaphoreType.DMA,
            pltpu.SemaphoreType.DMA,
            pltpu.SemaphoreType.DMA,
        ],
        compiler_params=pltpu.CompilerParams(needs_layout_passes=False),
        interpret=_INTERPRET,
    )(eid, rnk, bases, x)


def _sc_combine_body(dest_hbm, y_hbm, out_hbm, d0_v, d1_v, y0_v, y1_v,
                     sem_a, sem_b):
    hw = TPW // 2
    wid = lax.axis_index("s") * 2 + lax.axis_index("c")
    tok0 = wid * TPW
    ld0 = pltpu.async_copy(dest_hbm.at[pl.ds(tok0, hw)], d0_v, sem_a)
    ld1 = pltpu.async_copy(dest_hbm.at[pl.ds(tok0 + hw, hw)], d1_v, sem_b)
    ld0.wait()
    g0 = pltpu.async_copy(y_hbm.at[d0_v], y0_v, sem_a)
    ld1.wait()
    g1 = pltpu.async_copy(y_hbm.at[d1_v], y1_v, sem_b)
    g0.wait()
    pltpu.sync_copy(y0_v, out_hbm.at[pl.ds(tok0, hw)])
    g1.wait()
    pltpu.sync_copy(y1_v, out_hbm.at[pl.ds(tok0 + hw, hw)])


def _sc_combine(dest, y_sorted):
    mesh = plsc.VectorSubcoreMesh(core_axis_name="c", subcore_axis_name="s")
    return pl.kernel(
        _sc_combine_body,
        out_type=jax.ShapeDtypeStruct((T, D), jnp.float32),
        mesh=mesh,
        scratch_types=[
            pltpu.VMEM((TPW // 2,), jnp.int32),
            pltpu.VMEM((TPW // 2,), jnp.int32),
            pltpu.VMEM((TPW // 2, D), jnp.float32),
            pltpu.VMEM((TPW // 2, D), jnp.float32),
            pltpu.SemaphoreType.DMA,
            pltpu.SemaphoreType.DMA,
        ],
        compiler_params=pltpu.CompilerParams(needs_layout_passes=False),
        interpret=_INTERPRET,
    )(dest, y_sorted)


def _gmm_body(meta_ref, x_ref, w1_ref, b1_ref, w2_ref, b2_ref, y_ref):
    w = pl.program_id(0)

    @pl.when(w < meta_ref[0, 9])
    def _():
        t = meta_ref[1, w]
        e = meta_ref[2, w]
        s = meta_ref[0, e]
        epos = meta_ref[0, e + 1]
        rows = t * BT + jax.lax.broadcasted_iota(jnp.int32, (BT, 1), 0)
        mask = (rows >= s) & (rows < epos)
        h = jnp.maximum(
            jnp.dot(x_ref[...], w1_ref[0].astype(jnp.bfloat16),
                    preferred_element_type=jnp.float32) + b1_ref[0], 0.0)
        y = jnp.dot(h.astype(jnp.bfloat16), w2_ref[0].astype(jnp.bfloat16),
                    preferred_element_type=jnp.float32) + b2_ref[0]
        y_ref[...] = jnp.where(mask, y, y_ref[...])


def _gmm(meta, x_sorted, W1, b1, W2, b2):
    grid_spec = pltpu.PrefetchScalarGridSpec(
        num_scalar_prefetch=1,
        grid=(NW,),
        in_specs=[
            pl.BlockSpec((BT, D), lambda w, mr: (mr[1, w], 0)),
            pl.BlockSpec((1, D, H), lambda w, mr: (mr[2, w], 0, 0)),
            pl.BlockSpec((1, 1, H), lambda w, mr: (mr[2, w], 0, 0)),
            pl.BlockSpec((1, H, D), lambda w, mr: (mr[2, w], 0, 0)),
            pl.BlockSpec((1, 1, D), lambda w, mr: (mr[2, w], 0, 0)),
        ],
        out_specs=pl.BlockSpec((BT, D), lambda w, mr: (mr[1, w], 0)),
    )
    return pl.pallas_call(
        _gmm_body,
        grid_spec=grid_spec,
        out_shape=jax.ShapeDtypeStruct((T, D), jnp.float32),
        compiler_params=pltpu.CompilerParams(
            dimension_semantics=("arbitrary",)),
        interpret=_INTERPRET,
    )(meta, x_sorted, W1,
      b1.reshape(E, 1, H), W2, b2.reshape(E, 1, D))


def kernel(x, Wg, W1, b1, W2, b2):
    wg_pad = jnp.zeros((D, 128), jnp.float32).at[:, :E].set(Wg)
    eid, rnk, bases, meta, loss11 = _router(x, wg_pad)
    loss = loss11[0, 0]

    dest, x_sorted = _sc_dispatch(eid, rnk, bases, x)
    y_sorted = _gmm(meta, x_sorted, W1, b1, W2, b2)
    out = _sc_combine(dest, y_sorted)
    return out, loss, loss
